# baseline (speedup 1.0000x reference)
import functools

import jax
import jax.numpy as jnp
from jax import lax
from jax.experimental import pallas as pl
from jax.experimental.pallas import tpu as pltpu
from jax.experimental.pallas import tpu_sc as plsc

B = 4
NP = 10000
NNZ = 160000
N = B * NP

NC = 2
NS = 16
L = 16

BLK = 128
NB = NNZ // BLK
NB_TILE = 78
KP = 2
SWP = KP * BLK
NPASS = 39
TAIL_OFF = NS * NB_TILE * BLK
TAIL_W = NNZ - TAIL_OFF
NP_PAD = 10240
SLICE = NP_PAD // NS
MSE_CHUNK = 1280


def _sc_body(er_hbm, ei_hbm, ymr_hbm, ymi_hbm, rows_hbm, cols_hbm, vr_hbm, vi_hbm,
             br_hbm, bi_hbm,
             out_hbm,
             exch_hbm,
             str0, stc0, stvr0, stvi0, str1, stc1, stvr1, stvi1,
             str2, stc2, stvr2, stvi2,
             accr0, acci0, accr1, acci1, xr2, xi2,
             tmpa, tmpb, axbuf, m0, m1, m2, m3, brv, biv, outv,
             sem_st0, sem_st1, sem_st2, sem_x, sem_b, sem_mse, sem_t,
             sem_t2):
  c = lax.axis_index("c")
  s = lax.axis_index("s")

  zeros = jnp.zeros((L,), jnp.float32)
  iota = lax.broadcasted_iota(jnp.int32, (L,), 0)
  wid = c * NS + s

  st = [(str0, stc0, stvr0, stvi0, sem_st0),
        (str1, stc1, stvr1, stvi1, sem_st1),
        (str2, stc2, stvr2, stvi2, sem_st2)]
  accs = [(accr0, acci0), (accr1, acci1)]

  x_copies = [
      pltpu.async_copy(er_hbm.at[pl.ds(c * 2 * NP, 2 * NP)], xr2, sem_x),
      pltpu.async_copy(ei_hbm.at[pl.ds(c * 2 * NP, 2 * NP)], xi2, sem_x),
  ]
  mse_off = pl.multiple_of(jnp.minimum(wid * MSE_CHUNK, N - MSE_CHUNK), 8)
  mse_copies = [
      pltpu.async_copy(er_hbm.at[pl.ds(mse_off, MSE_CHUNK)], m0, sem_mse),
      pltpu.async_copy(ei_hbm.at[pl.ds(mse_off, MSE_CHUNK)], m1, sem_mse),
      pltpu.async_copy(ymr_hbm.at[pl.ds(mse_off, MSE_CHUNK)], m2, sem_mse),
      pltpu.async_copy(ymi_hbm.at[pl.ds(mse_off, MSE_CHUNK)], m3, sem_mse),
  ]

  col0 = s * (NB_TILE * BLK)

  def pass_copies(p, parity):
    r, co, vr_, vi_, sem = st[parity]
    off = pl.multiple_of(col0 + p * SWP, BLK)
    return [
        pltpu.make_async_copy(rows_hbm.at[:, pl.ds(off, SWP)], r, sem),
        pltpu.make_async_copy(cols_hbm.at[:, pl.ds(off, SWP)], co, sem),
        pltpu.make_async_copy(vr_hbm.at[:, pl.ds(off, SWP)], vr_, sem),
        pltpu.make_async_copy(vi_hbm.at[:, pl.ds(off, SWP)], vi_, sem),
    ]

  def fire_pass(p, parity):
    for cp in pass_copies(p, parity):
      cp.start()

  def wait_pass(p, parity):
    for cp in pass_copies(p, parity):
      cp.wait()

  fire_pass(0, 0)
  fire_pass(1, 1)

  @plsc.parallel_loop(0, NP // L, unroll=5)
  def _(k):
    off = pl.ds(k * L, L)
    accr0[off] = zeros
    acci0[off] = zeros
    accr1[off] = zeros
    acci1[off] = zeros

  with jax.named_scope("x_wait"):
    for cp in x_copies:
      cp.wait()

  def phase1_block(r_ref, c_ref, vr_ref, vi_ref, nvregs):
    for b_local in range(2):
      brow16 = jnp.full((L,), 2 * c + b_local, jnp.int32)
      acc_r, acc_i = accs[b_local]
      xbase = b_local * NP

      @plsc.parallel_loop(0, nvregs, unroll=4)
      def _(t):
        idx16 = t * L + iota
        rowg = plsc.load_gather(r_ref, [brow16, idx16])
        colg = plsc.load_gather(c_ref, [brow16, idx16])
        wr = plsc.load_gather(vr_ref, [brow16, idx16])
        wi = plsc.load_gather(vi_ref, [brow16, idx16])
        xcr = plsc.load_gather(xr2, [colg + xbase])
        xci = plsc.load_gather(xi2, [colg + xbase])
        ar = wr * xcr - wi * xci
        ai = wr * xci + wi * xcr
        plsc.addupdate_scatter(acc_r, [rowg], ar)
        plsc.addupdate_scatter(acc_i, [rowg], ai)

  def pass_triple(k, _):
    p0 = k * 3
    for j in range(3):
      p = p0 + j

      @pl.when(p + 2 < NPASS)
      def _():
        fire_pass(p + 2, (j + 2) % 3)

      with jax.named_scope("st_wait"):
        wait_pass(p, j)
      r_ref, c_ref, vr_ref, vi_ref, _ = st[j]
      with jax.named_scope("phase1"):
        phase1_block(r_ref, c_ref, vr_ref, vi_ref, SWP // L)
    return 0

  lax.fori_loop(0, NPASS // 3, pass_triple, 0)

  @pl.when(s == 0)
  def _():
    pltpu.sync_copy(rows_hbm.at[:, pl.ds(TAIL_OFF, TAIL_W)],
                    str0.at[:, pl.ds(0, TAIL_W)])
    pltpu.sync_copy(cols_hbm.at[:, pl.ds(TAIL_OFF, TAIL_W)],
                    stc0.at[:, pl.ds(0, TAIL_W)])
    pltpu.sync_copy(vr_hbm.at[:, pl.ds(TAIL_OFF, TAIL_W)],
                    stvr0.at[:, pl.ds(0, TAIL_W)])
    pltpu.sync_copy(vi_hbm.at[:, pl.ds(TAIL_OFF, TAIL_W)],
                    stvi0.at[:, pl.ds(0, TAIL_W)])
    phase1_block(str0, stc0, stvr0, stvi0, TAIL_W // L)

  row_base = s * SLICE
  b_off = pl.multiple_of(jnp.minimum(row_base, NP - SLICE), 8)

  def b_slices(bi):
    boff = pl.multiple_of(bi * NP, 8) + b_off
    return (br_hbm.at[pl.ds(boff, SLICE)], bi_hbm.at[pl.ds(boff, SLICE)])

  def pub_off(tile, slot):
    return pl.multiple_of(((c * NS + tile) * 4 + slot) * NP, 8)

  with jax.named_scope("publish"):
    pubs = [
        pltpu.async_copy(accr0, exch_hbm.at[pl.ds(pub_off(s, 0), NP)], sem_b),
        pltpu.async_copy(acci0, exch_hbm.at[pl.ds(pub_off(s, 1), NP)], sem_b),
        pltpu.async_copy(accr1, exch_hbm.at[pl.ds(pub_off(s, 2), NP)], sem_b),
        pltpu.async_copy(acci1, exch_hbm.at[pl.ds(pub_off(s, 3), NP)], sem_b),
    ]
    for cp in pubs:
      cp.wait()
    plsc.subcore_barrier()

  b_shift = row_base - b_off
  loss_vec = zeros
  pairs = [(q, h) for q in range(4) for h in range(2)]
  tmps = [tmpa, tmpb]
  semt = [sem_t, sem_t2]

  def pair_copies(i):
    q, h = pairs[i]
    buf = tmps[i % 2]
    return [
        pltpu.make_async_copy(
            exch_hbm.at[pl.ds(
                pl.multiple_of(pub_off(h * 8 + j, q) + b_off, 8), SLICE)],
            buf.at[pl.ds(j * SLICE, SLICE)], semt[i % 2])
        for j in range(8)
    ]

  for cp in pair_copies(0):
    cp.start()
  bcp = [pltpu.async_copy(b_slices(2 * c)[0], brv, sem_b),
         pltpu.async_copy(b_slices(2 * c)[1], biv, sem_b)]

  for i, (q, h) in enumerate(pairs):
    if i + 1 < len(pairs):
      for cp in pair_copies(i + 1):
        cp.start()
    with jax.named_scope("tmp_copy"):
      for cp in pair_copies(i):
        cp.wait()
    buf = tmps[i % 2]
    b_ref = brv if q % 2 == 0 else biv

    if h == 0:
      @plsc.parallel_loop(0, SLICE // L)
      def half0(k):
        koff = jnp.minimum(b_shift + k * L, SLICE - L)
        part = buf[pl.ds(koff, L)]
        for j in range(1, 8):
          part = part + buf[pl.ds(j * SLICE + koff, L)]
        axbuf[pl.ds(k * L, L)] = part
    else:
      if q == 1:
        pass
      if q == 0:
        with jax.named_scope("b_wait"):
          for cp in bcp:
            cp.wait()

      @plsc.parallel_loop(0, SLICE // L, carry=loss_vec)
      def half1(k, acc):
        koff = jnp.minimum(b_shift + k * L, SLICE - L)
        part = buf[pl.ds(koff, L)]
        for j in range(1, 8):
          part = part + buf[pl.ds(j * SLICE + koff, L)]
        ax = axbuf[pl.ds(k * L, L)] + part
        r = b_ref[pl.ds(koff, L)] - ax
        row_vec = row_base + k * L + iota
        return acc + jnp.where(row_vec < NP, r * r, jnp.float32(0.0))
      loss_vec = half1

      if q == 1:
        bcp = [pltpu.async_copy(b_slices(2 * c + 1)[0], brv, sem_b),
               pltpu.async_copy(b_slices(2 * c + 1)[1], biv, sem_b)]
        with jax.named_scope("b_wait"):
          for cp in bcp:
            cp.wait()

  mse_shift = wid * MSE_CHUNK - mse_off
  with jax.named_scope("mse_wait"):
    for cp in mse_copies:
      cp.wait()

  @plsc.parallel_loop(0, MSE_CHUNK // L, carry=loss_vec)
  def mse_body(k, acc):
    roff = jnp.minimum(mse_shift + k * L, MSE_CHUNK - L)
    off = pl.ds(roff, L)
    dr = m0[off] - m2[off]
    di = m1[off] - m3[off]
    contrib = dr * dr + di * di
    elem = wid * MSE_CHUNK + k * L + iota
    return acc + jnp.where(elem < N, contrib, jnp.float32(0.0))
  loss_vec = mse_body

  outv[...] = loss_vec * jnp.float32(0.5 / N)
  pltpu.sync_copy(outv, out_hbm.at[wid])


@jax.jit
def _run(er, ei, ymr, ymi, rows, cols, vr, vi, br, bi):
  mesh = plsc.VectorSubcoreMesh(
      core_axis_name="c", subcore_axis_name="s",
      num_cores=NC, num_subcores=NS)
  f = pl.kernel(
      _sc_body,
      out_type=jax.ShapeDtypeStruct((NC * NS, L), jnp.float32),
      mesh=mesh,
      compiler_params=pltpu.CompilerParams(
          needs_layout_passes=False, skip_device_barrier=True,
          disable_bounds_checks=True),
      scratch_types=[
          pltpu.HBM((NC * NS * 4 * NP,), jnp.float32),
          pltpu.VMEM((B, SWP), jnp.int32),
          pltpu.VMEM((B, SWP), jnp.int32),
          pltpu.VMEM((B, SWP), jnp.float32),
          pltpu.VMEM((B, SWP), jnp.float32),
          pltpu.VMEM((B, SWP), jnp.int32),
          pltpu.VMEM((B, SWP), jnp.int32),
          pltpu.VMEM((B, SWP), jnp.float32),
          pltpu.VMEM((B, SWP), jnp.float32),
          pltpu.VMEM((B, SWP), jnp.int32),
          pltpu.VMEM((B, SWP), jnp.int32),
          pltpu.VMEM((B, SWP), jnp.float32),
          pltpu.VMEM((B, SWP), jnp.float32),
          pltpu.VMEM((NP,), jnp.float32),
          pltpu.VMEM((NP,), jnp.float32),
          pltpu.VMEM((NP,), jnp.float32),
          pltpu.VMEM((NP,), jnp.float32),
          pltpu.VMEM((2 * NP,), jnp.float32),
          pltpu.VMEM((2 * NP,), jnp.float32),
          pltpu.VMEM((8 * SLICE,), jnp.float32),
          pltpu.VMEM((8 * SLICE,), jnp.float32),
          pltpu.VMEM((SLICE,), jnp.float32),
          pltpu.VMEM((MSE_CHUNK,), jnp.float32),
          pltpu.VMEM((MSE_CHUNK,), jnp.float32),
          pltpu.VMEM((MSE_CHUNK,), jnp.float32),
          pltpu.VMEM((MSE_CHUNK,), jnp.float32),
          pltpu.VMEM((SLICE,), jnp.float32),
          pltpu.VMEM((SLICE,), jnp.float32),
          pltpu.VMEM((L,), jnp.float32),
          pltpu.SemaphoreType.DMA,
          pltpu.SemaphoreType.DMA,
          pltpu.SemaphoreType.DMA,
          pltpu.SemaphoreType.DMA,
          pltpu.SemaphoreType.DMA,
          pltpu.SemaphoreType.DMA,
          pltpu.SemaphoreType.DMA,
          pltpu.SemaphoreType.DMA,
      ],
  )
  return f(er, ei, ymr, ymi, rows, cols, vr, vi, br, bi)


def kernel(E_real, E_imag, batch_y, k_all, node_batch, A_rows, A_cols,
           A_vals_real, A_vals_imag, b_real, b_imag):
  del k_all, node_batch
  partials = _run(E_real, E_imag, batch_y[:, 0], batch_y[:, 1],
                  A_rows, A_cols, A_vals_real, A_vals_imag,
                  b_real.reshape(-1), b_imag.reshape(-1))
  return jnp.sum(partials)


# speedup vs baseline: 1.0101x; 1.0101x over previous
import functools

import jax
import jax.numpy as jnp
from jax import lax
from jax.experimental import pallas as pl
from jax.experimental.pallas import tpu as pltpu
from jax.experimental.pallas import tpu_sc as plsc

B = 4
NP = 10000
NNZ = 160000
N = B * NP

NC = 2
NS = 16
L = 16

BLK = 128
NB = NNZ // BLK
NB_TILE = 78
KP = 2
SWP = KP * BLK
NPASS = 39
NRING = 3
TAIL_OFF = NS * NB_TILE * BLK
TAIL_W = NNZ - TAIL_OFF
NP_PAD = 10240
SLICE = NP_PAD // NS
MSE_CHUNK = 1280
NV = SWP // L


def _sc_body(er_hbm, ei_hbm, ymr_hbm, ymi_hbm, rows_hbm, cols_hbm,
             vr_hbm, vi_hbm, br_hbm, bi_hbm,
             out_hbm, exch_hbm,
             str0, stc0, stvr0, stvi0, str1, stc1, stvr1, stvi1,
             str2, stc2, stvr2, stvi2,
             accr, acci, xr2, xi2,
             tmpa, tmpb, axbuf, b4, m0, m1, m2, m3, outv,
             sem_st0, sem_st1, sem_st2, sem_x, sem_b, sem_mse,
             sem_t, sem_t2):
  c = lax.axis_index("c")
  s = lax.axis_index("s")

  zeros = jnp.zeros((L,), jnp.float32)
  iota = lax.broadcasted_iota(jnp.int32, (L,), 0)
  wid = c * NS + s

  st = [(str0, stc0, stvr0, stvi0, sem_st0),
        (str1, stc1, stvr1, stvi1, sem_st1),
        (str2, stc2, stvr2, stvi2, sem_st2)]

  x_copies = [
      pltpu.async_copy(er_hbm.at[pl.ds(c * 2 * NP, 2 * NP)], xr2, sem_x),
      pltpu.async_copy(ei_hbm.at[pl.ds(c * 2 * NP, 2 * NP)], xi2, sem_x),
  ]
  mse_off = pl.multiple_of(jnp.minimum(wid * MSE_CHUNK, N - MSE_CHUNK), 8)
  mse_copies = [
      pltpu.async_copy(er_hbm.at[pl.ds(mse_off, MSE_CHUNK)], m0, sem_mse),
      pltpu.async_copy(ei_hbm.at[pl.ds(mse_off, MSE_CHUNK)], m1, sem_mse),
      pltpu.async_copy(ymr_hbm.at[pl.ds(mse_off, MSE_CHUNK)], m2, sem_mse),
      pltpu.async_copy(ymi_hbm.at[pl.ds(mse_off, MSE_CHUNK)], m3, sem_mse),
  ]

  col0 = s * (NB_TILE * BLK)

  def pass_copies(p, parity):
    r, co, vr_, vi_, sem = st[parity]
    off = pl.multiple_of(col0 + p * SWP, BLK)
    return [
        pltpu.make_async_copy(rows_hbm.at[:, pl.ds(off, SWP)], r, sem),
        pltpu.make_async_copy(cols_hbm.at[:, pl.ds(off, SWP)], co, sem),
        pltpu.make_async_copy(vr_hbm.at[:, pl.ds(off, SWP)], vr_, sem),
        pltpu.make_async_copy(vi_hbm.at[:, pl.ds(off, SWP)], vi_, sem),
    ]

  def fire_pass(p, parity):
    for cp in pass_copies(p, parity):
      cp.start()

  def wait_pass(p, parity):
    for cp in pass_copies(p, parity):
      cp.wait()

  fire_pass(0, 0)
  fire_pass(1, 1)

  @plsc.parallel_loop(0, 2 * NP // L, unroll=5)
  def _(k):
    off = pl.ds(k * L, L)
    accr[off] = zeros
    acci[off] = zeros

  with jax.named_scope("x_wait"):
    for cp in x_copies:
      cp.wait()

  brow16_0 = jnp.full((L,), 2 * c, jnp.int32)

  def phase1_block(r_ref, c_ref, vr_ref, vi_ref, nvregs):
    @plsc.parallel_loop(0, 2 * nvregs, unroll=4)
    def _(t):
      second = t >= nvregs
      idx16 = jnp.where(second, t - nvregs, t) * L + iota
      xoff = jnp.where(second, NP, 0)
      brow16 = brow16_0 + second.astype(jnp.int32)
      rowg = plsc.load_gather(r_ref, [brow16, idx16])
      colg = plsc.load_gather(c_ref, [brow16, idx16])
      wr = plsc.load_gather(vr_ref, [brow16, idx16])
      wi = plsc.load_gather(vi_ref, [brow16, idx16])
      xcr = plsc.load_gather(xr2, [colg + xoff])
      xci = plsc.load_gather(xi2, [colg + xoff])
      ar = wr * xcr - wi * xci
      ai = wr * xci + wi * xcr
      plsc.addupdate_scatter(accr, [rowg + xoff], ar)
      plsc.addupdate_scatter(acci, [rowg + xoff], ai)

  def pass_triple(k, _):
    p0 = k * 3
    for j in range(NRING):
      p = p0 + j

      @pl.when(p + 2 < NPASS)
      def _():
        fire_pass(p + 2, (j + 2) % NRING)

      with jax.named_scope("st_wait"):
        wait_pass(p, j)
      r_ref, c_ref, vr_ref, vi_ref, _ = st[j]
      with jax.named_scope("phase1"):
        phase1_block(r_ref, c_ref, vr_ref, vi_ref, NV)
    return 0

  lax.fori_loop(0, NPASS // NRING, pass_triple, 0)

  @pl.when(s == 0)
  def _():
    pltpu.sync_copy(rows_hbm.at[:, pl.ds(TAIL_OFF, TAIL_W)], str0)
    pltpu.sync_copy(cols_hbm.at[:, pl.ds(TAIL_OFF, TAIL_W)], stc0)
    pltpu.sync_copy(vr_hbm.at[:, pl.ds(TAIL_OFF, TAIL_W)], stvr0)
    pltpu.sync_copy(vi_hbm.at[:, pl.ds(TAIL_OFF, TAIL_W)], stvi0)
    phase1_block(str0, stc0, stvr0, stvi0, TAIL_W // L)

  row_base = s * SLICE
  b_off = pl.multiple_of(jnp.minimum(row_base, NP - SLICE), 8)
  b_shift = row_base - b_off
  b_srcs = [br_hbm.at[pl.ds(pl.multiple_of(2 * c * NP, 8) + b_off, SLICE)],
            bi_hbm.at[pl.ds(pl.multiple_of(2 * c * NP, 8) + b_off, SLICE)],
            br_hbm.at[pl.ds(pl.multiple_of((2 * c + 1) * NP, 8) + b_off,
                            SLICE)],
            bi_hbm.at[pl.ds(pl.multiple_of((2 * c + 1) * NP, 8) + b_off,
                            SLICE)]]
  bcp = [pltpu.async_copy(src, b4.at[pl.ds(q * SLICE, SLICE)], sem_b)
         for q, src in enumerate(b_srcs)]

  def pub_off(tile, slot):
    return pl.multiple_of(((c * NS + tile) * 4 + slot) * NP, 8)

  with jax.named_scope("publish"):
    pubs0 = [
        pltpu.async_copy(accr.at[pl.ds(0, NP)],
                         exch_hbm.at[pl.ds(pub_off(s, 0), NP)], sem_x),
        pltpu.async_copy(acci.at[pl.ds(0, NP)],
                         exch_hbm.at[pl.ds(pub_off(s, 1), NP)], sem_x),
    ]
    pubs1 = [
        pltpu.async_copy(accr.at[pl.ds(NP, NP)],
                         exch_hbm.at[pl.ds(pub_off(s, 2), NP)], sem_st0),
        pltpu.async_copy(acci.at[pl.ds(NP, NP)],
                         exch_hbm.at[pl.ds(pub_off(s, 3), NP)], sem_st0),
    ]
    for cp in pubs0:
      cp.wait()
    plsc.subcore_barrier()

  def half_copies(q, h, buf, sem):
    return [
        pltpu.make_async_copy(
            exch_hbm.at[pl.ds(
                pl.multiple_of(pub_off(h * 8 + j, q) + b_off, 8), SLICE)],
            buf.at[pl.ds(j * SLICE, SLICE)], sem)
        for j in range(8)
    ]

  def fire_half(q, h, buf, sem):
    for cp in half_copies(q, h, buf, sem):
      cp.start()

  def wait_half(q, h, buf, sem):
    for cp in half_copies(q, h, buf, sem):
      cp.wait()

  fire_half(0, 0, tmpa, sem_t)
  fire_half(0, 1, tmpb, sem_t2)

  loss_vec = zeros

  def slot_round(q, acc):
    with jax.named_scope("tmp_wait"):
      wait_half(q, 0, tmpa, sem_t)

    @plsc.parallel_loop(0, SLICE // L)
    def half0(k):
      koff = jnp.minimum(b_shift + k * L, SLICE - L)
      part = tmpa[pl.ds(koff, L)]
      for j in range(1, 8):
        part = part + tmpa[pl.ds(j * SLICE + koff, L)]
      axbuf[pl.ds(k * L, L)] = part

    @pl.when(q < 3)
    def _():
      fire_half(q + 1, 0, tmpa, sem_t)

    with jax.named_scope("tmp_wait"):
      wait_half(q, 1, tmpb, sem_t2)

    @plsc.parallel_loop(0, SLICE // L, carry=acc)
    def half1(k, a):
      koff = jnp.minimum(b_shift + k * L, SLICE - L)
      part = tmpb[pl.ds(koff, L)]
      for j in range(1, 8):
        part = part + tmpb[pl.ds(j * SLICE + koff, L)]
      ax = axbuf[pl.ds(k * L, L)] + part
      r = b4[pl.ds(q * SLICE + koff, L)] - ax
      row_vec = row_base + k * L + iota
      return a + jnp.where(row_vec < NP, r * r, jnp.float32(0.0))

    @pl.when(q < 3)
    def _():
      fire_half(q + 1, 1, tmpb, sem_t2)

    return half1

  with jax.named_scope("b_wait"):
    for cp in bcp:
      cp.wait()
  loss_vec = slot_round(0, loss_vec)
  with jax.named_scope("publish"):
    for cp in pubs1:
      cp.wait()
    plsc.subcore_barrier()
  loss_vec = slot_round(1, loss_vec)
  loss_vec = slot_round(2, loss_vec)
  loss_vec = slot_round(3, loss_vec)

  mse_shift = wid * MSE_CHUNK - mse_off
  with jax.named_scope("mse_wait"):
    for cp in mse_copies:
      cp.wait()

  @plsc.parallel_loop(0, MSE_CHUNK // L, carry=loss_vec)
  def mse_body(k, acc):
    roff = jnp.minimum(mse_shift + k * L, MSE_CHUNK - L)
    off = pl.ds(roff, L)
    dr = m0[off] - m2[off]
    di = m1[off] - m3[off]
    contrib = dr * dr + di * di
    elem = wid * MSE_CHUNK + k * L + iota
    return acc + jnp.where(elem < N, contrib, jnp.float32(0.0))
  loss_vec = mse_body

  outv[...] = loss_vec * jnp.float32(0.5 / N)
  pltpu.sync_copy(outv, out_hbm.at[wid])


@jax.jit
def _run(er, ei, ymr, ymi, rows, cols, vr, vi, br, bi):
  mesh = plsc.VectorSubcoreMesh(
      core_axis_name="c", subcore_axis_name="s",
      num_cores=NC, num_subcores=NS)
  f = pl.kernel(
      _sc_body,
      out_type=jax.ShapeDtypeStruct((NC * NS, L), jnp.float32),
      mesh=mesh,
      compiler_params=pltpu.CompilerParams(
          needs_layout_passes=False, skip_device_barrier=True,
          disable_bounds_checks=True),
      scratch_types=[
          pltpu.HBM((NC * NS * 4 * NP,), jnp.float32),
          pltpu.VMEM((B, SWP), jnp.int32),
          pltpu.VMEM((B, SWP), jnp.int32),
          pltpu.VMEM((B, SWP), jnp.float32),
          pltpu.VMEM((B, SWP), jnp.float32),
          pltpu.VMEM((B, SWP), jnp.int32),
          pltpu.VMEM((B, SWP), jnp.int32),
          pltpu.VMEM((B, SWP), jnp.float32),
          pltpu.VMEM((B, SWP), jnp.float32),
          pltpu.VMEM((B, SWP), jnp.int32),
          pltpu.VMEM((B, SWP), jnp.int32),
          pltpu.VMEM((B, SWP), jnp.float32),
          pltpu.VMEM((B, SWP), jnp.float32),
          pltpu.VMEM((2 * NP,), jnp.float32),
          pltpu.VMEM((2 * NP,), jnp.float32),
          pltpu.VMEM((2 * NP,), jnp.float32),
          pltpu.VMEM((2 * NP,), jnp.float32),
          pltpu.VMEM((8 * SLICE,), jnp.float32),
          pltpu.VMEM((8 * SLICE,), jnp.float32),
          pltpu.VMEM((SLICE,), jnp.float32),
          pltpu.VMEM((4 * SLICE,), jnp.float32),
          pltpu.VMEM((MSE_CHUNK,), jnp.float32),
          pltpu.VMEM((MSE_CHUNK,), jnp.float32),
          pltpu.VMEM((MSE_CHUNK,), jnp.float32),
          pltpu.VMEM((MSE_CHUNK,), jnp.float32),
          pltpu.VMEM((L,), jnp.float32),
          pltpu.SemaphoreType.DMA,
          pltpu.SemaphoreType.DMA,
          pltpu.SemaphoreType.DMA,
          pltpu.SemaphoreType.DMA,
          pltpu.SemaphoreType.DMA,
          pltpu.SemaphoreType.DMA,
          pltpu.SemaphoreType.DMA,
          pltpu.SemaphoreType.DMA,
      ],
  )
  return f(er, ei, ymr, ymi, rows, cols, vr, vi, br, bi)


def kernel(E_real, E_imag, batch_y, k_all, node_batch, A_rows, A_cols,
           A_vals_real, A_vals_imag, b_real, b_imag):
  del k_all, node_batch
  partials = _run(E_real, E_imag, batch_y[:, 0], batch_y[:, 1],
                  A_rows, A_cols, A_vals_real, A_vals_imag,
                  b_real.reshape(-1), b_imag.reshape(-1))
  return jnp.sum(partials)


# speedup vs baseline: 1.0517x; 1.0412x over previous
import functools

import jax
import jax.numpy as jnp
from jax import lax
from jax.experimental import pallas as pl
from jax.experimental.pallas import tpu as pltpu
from jax.experimental.pallas import tpu_sc as plsc

B = 4
NP = 10000
NNZ = 160000
N = B * NP

NC = 2
NS = 16
L = 16

BLK = 128
NB = NNZ // BLK
NB_TILE = 78
KP = 2
SWP = KP * BLK
NPASS = 39
NRING = 3
TAIL_OFF = NS * NB_TILE * BLK
TAIL_W = NNZ - TAIL_OFF
NP_PAD = 10240
SLICE = NP_PAD // NS
MSE_CHUNK = 1280
NV = SWP // L


def _sc_body(er_hbm, ei_hbm, ymr_hbm, ymi_hbm, rows_hbm, cols_hbm,
             vr_hbm, vi_hbm, br_hbm, bi_hbm,
             out_hbm, exch_hbm,
             str0, stc0, stvr0, stvi0, str1, stc1, stvr1, stvi1,
             str2, stc2, stvr2, stvi2,
             accr, acci, xr2, xi2,
             tmpa, tmpb, axbuf, b4, m0, m1, m2, m3, outv, shx,
             sem_st0, sem_st1, sem_st2, sem_x, sem_b, sem_mse,
             sem_t, sem_t2):
  c = lax.axis_index("c")
  s = lax.axis_index("s")

  zeros = jnp.zeros((L,), jnp.float32)
  iota = lax.broadcasted_iota(jnp.int32, (L,), 0)
  wid = c * NS + s

  st = [(str0, stc0, stvr0, stvi0, sem_st0),
        (str1, stc1, stvr1, stvi1, sem_st1),
        (str2, stc2, stvr2, stvi2, sem_st2)]

  @pl.when(s == 0)
  def _():
    pltpu.async_copy(er_hbm.at[pl.ds(c * 2 * NP, 2 * NP)], xr2,
                     sem_x).start()
    pltpu.async_copy(ei_hbm.at[pl.ds(c * 2 * NP, 2 * NP)], xi2,
                     sem_x).start()
  mse_off = pl.multiple_of(jnp.minimum(wid * MSE_CHUNK, N - MSE_CHUNK), 8)
  mse_copies = [
      pltpu.async_copy(er_hbm.at[pl.ds(mse_off, MSE_CHUNK)], m0, sem_mse),
      pltpu.async_copy(ei_hbm.at[pl.ds(mse_off, MSE_CHUNK)], m1, sem_mse),
      pltpu.async_copy(ymr_hbm.at[pl.ds(mse_off, MSE_CHUNK)], m2, sem_mse),
      pltpu.async_copy(ymi_hbm.at[pl.ds(mse_off, MSE_CHUNK)], m3, sem_mse),
  ]

  col0 = s * (NB_TILE * BLK)

  def pass_copies(p, parity):
    r, co, vr_, vi_, sem = st[parity]
    off = pl.multiple_of(col0 + p * SWP, BLK)
    return [
        pltpu.make_async_copy(rows_hbm.at[:, pl.ds(off, SWP)], r, sem),
        pltpu.make_async_copy(cols_hbm.at[:, pl.ds(off, SWP)], co, sem),
        pltpu.make_async_copy(vr_hbm.at[:, pl.ds(off, SWP)], vr_, sem),
        pltpu.make_async_copy(vi_hbm.at[:, pl.ds(off, SWP)], vi_, sem),
    ]

  def fire_pass(p, parity):
    for cp in pass_copies(p, parity):
      cp.start()

  def wait_pass(p, parity):
    for cp in pass_copies(p, parity):
      cp.wait()

  fire_pass(0, 0)
  fire_pass(1, 1)

  @plsc.parallel_loop(0, 2 * NP // L, unroll=5)
  def _(k):
    off = pl.ds(k * L, L)
    accr[off] = zeros
    acci[off] = zeros

  with jax.named_scope("x_wait"):
    @pl.when(s == 0)
    def _():
      pltpu.make_async_copy(er_hbm.at[pl.ds(c * 2 * NP, 2 * NP)], xr2,
                            sem_x).wait()
      pltpu.make_async_copy(ei_hbm.at[pl.ds(c * 2 * NP, 2 * NP)], xi2,
                            sem_x).wait()
      pltpu.sync_copy(xr2, shx.at[pl.ds(0, 2 * NP)])
      pltpu.sync_copy(xi2, shx.at[pl.ds(2 * NP, 2 * NP)])
    plsc.subcore_barrier()

    @pl.when(s != 0)
    def _():
      pltpu.sync_copy(shx.at[pl.ds(0, 2 * NP)], xr2)
      pltpu.sync_copy(shx.at[pl.ds(2 * NP, 2 * NP)], xi2)

  brow16_0 = jnp.full((L,), 2 * c, jnp.int32)

  def phase1_block(r_ref, c_ref, vr_ref, vi_ref, nvregs):
    @plsc.parallel_loop(0, 2 * nvregs, unroll=4)
    def _(t):
      second = t >= nvregs
      idx16 = jnp.where(second, t - nvregs, t) * L + iota
      xoff = jnp.where(second, NP, 0)
      brow16 = brow16_0 + second.astype(jnp.int32)
      rowg = plsc.load_gather(r_ref, [brow16, idx16])
      colg = plsc.load_gather(c_ref, [brow16, idx16])
      wr = plsc.load_gather(vr_ref, [brow16, idx16])
      wi = plsc.load_gather(vi_ref, [brow16, idx16])
      xcr = plsc.load_gather(xr2, [colg + xoff])
      xci = plsc.load_gather(xi2, [colg + xoff])
      ar = wr * xcr - wi * xci
      ai = wr * xci + wi * xcr
      plsc.addupdate_scatter(accr, [rowg + xoff], ar)
      plsc.addupdate_scatter(acci, [rowg + xoff], ai)

  def pass_triple(k, _):
    p0 = k * 3
    for j in range(NRING):
      p = p0 + j

      @pl.when(p + 2 < NPASS)
      def _():
        fire_pass(p + 2, (j + 2) % NRING)

      with jax.named_scope("st_wait"):
        wait_pass(p, j)
      r_ref, c_ref, vr_ref, vi_ref, _ = st[j]
      with jax.named_scope("phase1"):
        phase1_block(r_ref, c_ref, vr_ref, vi_ref, NV)
    return 0

  lax.fori_loop(0, NPASS // NRING, pass_triple, 0)

  @pl.when(s == 0)
  def _():
    pltpu.sync_copy(rows_hbm.at[:, pl.ds(TAIL_OFF, TAIL_W)], str0)
    pltpu.sync_copy(cols_hbm.at[:, pl.ds(TAIL_OFF, TAIL_W)], stc0)
    pltpu.sync_copy(vr_hbm.at[:, pl.ds(TAIL_OFF, TAIL_W)], stvr0)
    pltpu.sync_copy(vi_hbm.at[:, pl.ds(TAIL_OFF, TAIL_W)], stvi0)
    phase1_block(str0, stc0, stvr0, stvi0, TAIL_W // L)

  row_base = s * SLICE
  b_off = pl.multiple_of(jnp.minimum(row_base, NP - SLICE), 8)
  b_shift = row_base - b_off
  b_srcs = [br_hbm.at[pl.ds(pl.multiple_of(2 * c * NP, 8) + b_off, SLICE)],
            bi_hbm.at[pl.ds(pl.multiple_of(2 * c * NP, 8) + b_off, SLICE)],
            br_hbm.at[pl.ds(pl.multiple_of((2 * c + 1) * NP, 8) + b_off,
                            SLICE)],
            bi_hbm.at[pl.ds(pl.multiple_of((2 * c + 1) * NP, 8) + b_off,
                            SLICE)]]
  bcp = [pltpu.async_copy(src, b4.at[pl.ds(q * SLICE, SLICE)], sem_b)
         for q, src in enumerate(b_srcs)]

  def pub_off(tile, slot):
    return pl.multiple_of(((c * NS + tile) * 4 + slot) * NP, 8)

  with jax.named_scope("publish"):
    pubs0 = [
        pltpu.async_copy(accr.at[pl.ds(0, NP)],
                         exch_hbm.at[pl.ds(pub_off(s, 0), NP)], sem_x),
        pltpu.async_copy(acci.at[pl.ds(0, NP)],
                         exch_hbm.at[pl.ds(pub_off(s, 1), NP)], sem_x),
    ]
    pubs1 = [
        pltpu.async_copy(accr.at[pl.ds(NP, NP)],
                         exch_hbm.at[pl.ds(pub_off(s, 2), NP)], sem_st0),
        pltpu.async_copy(acci.at[pl.ds(NP, NP)],
                         exch_hbm.at[pl.ds(pub_off(s, 3), NP)], sem_st0),
    ]
    for cp in pubs0:
      cp.wait()
    plsc.subcore_barrier()

  def half_copies(q, h, buf, sem):
    return [
        pltpu.make_async_copy(
            exch_hbm.at[pl.ds(
                pl.multiple_of(pub_off(h * 8 + j, q) + b_off, 8), SLICE)],
            buf.at[pl.ds(j * SLICE, SLICE)], sem)
        for j in range(8)
    ]

  def fire_half(q, h, buf, sem):
    for cp in half_copies(q, h, buf, sem):
      cp.start()

  def wait_half(q, h, buf, sem):
    for cp in half_copies(q, h, buf, sem):
      cp.wait()

  fire_half(0, 0, tmpa, sem_t)
  fire_half(0, 1, tmpb, sem_t2)

  loss_vec = zeros

  def slot_round(q, acc):
    with jax.named_scope("tmp_wait"):
      wait_half(q, 0, tmpa, sem_t)

    @plsc.parallel_loop(0, SLICE // L)
    def half0(k):
      koff = jnp.minimum(b_shift + k * L, SLICE - L)
      part = tmpa[pl.ds(koff, L)]
      for j in range(1, 8):
        part = part + tmpa[pl.ds(j * SLICE + koff, L)]
      axbuf[pl.ds(k * L, L)] = part

    @pl.when(q < 3)
    def _():
      fire_half(q + 1, 0, tmpa, sem_t)

    with jax.named_scope("tmp_wait"):
      wait_half(q, 1, tmpb, sem_t2)

    @plsc.parallel_loop(0, SLICE // L, carry=acc)
    def half1(k, a):
      koff = jnp.minimum(b_shift + k * L, SLICE - L)
      part = tmpb[pl.ds(koff, L)]
      for j in range(1, 8):
        part = part + tmpb[pl.ds(j * SLICE + koff, L)]
      ax = axbuf[pl.ds(k * L, L)] + part
      r = b4[pl.ds(q * SLICE + koff, L)] - ax
      row_vec = row_base + k * L + iota
      return a + jnp.where(row_vec < NP, r * r, jnp.float32(0.0))

    @pl.when(q < 3)
    def _():
      fire_half(q + 1, 1, tmpb, sem_t2)

    return half1

  with jax.named_scope("b_wait"):
    for cp in bcp:
      cp.wait()
  loss_vec = slot_round(0, loss_vec)
  with jax.named_scope("publish"):
    for cp in pubs1:
      cp.wait()
    plsc.subcore_barrier()
  loss_vec = slot_round(1, loss_vec)
  loss_vec = slot_round(2, loss_vec)
  loss_vec = slot_round(3, loss_vec)

  mse_shift = wid * MSE_CHUNK - mse_off
  with jax.named_scope("mse_wait"):
    for cp in mse_copies:
      cp.wait()

  @plsc.parallel_loop(0, MSE_CHUNK // L, carry=loss_vec)
  def mse_body(k, acc):
    roff = jnp.minimum(mse_shift + k * L, MSE_CHUNK - L)
    off = pl.ds(roff, L)
    dr = m0[off] - m2[off]
    di = m1[off] - m3[off]
    contrib = dr * dr + di * di
    elem = wid * MSE_CHUNK + k * L + iota
    return acc + jnp.where(elem < N, contrib, jnp.float32(0.0))
  loss_vec = mse_body

  outv[...] = loss_vec * jnp.float32(0.5 / N)
  pltpu.sync_copy(outv, out_hbm.at[wid])


@jax.jit
def _run(er, ei, ymr, ymi, rows, cols, vr, vi, br, bi):
  mesh = plsc.VectorSubcoreMesh(
      core_axis_name="c", subcore_axis_name="s",
      num_cores=NC, num_subcores=NS)
  f = pl.kernel(
      _sc_body,
      out_type=jax.ShapeDtypeStruct((NC * NS, L), jnp.float32),
      mesh=mesh,
      compiler_params=pltpu.CompilerParams(
          needs_layout_passes=False, skip_device_barrier=True,
          disable_bounds_checks=True),
      scratch_types=[
          pltpu.HBM((NC * NS * 4 * NP,), jnp.float32),
          pltpu.VMEM((B, SWP), jnp.int32),
          pltpu.VMEM((B, SWP), jnp.int32),
          pltpu.VMEM((B, SWP), jnp.float32),
          pltpu.VMEM((B, SWP), jnp.float32),
          pltpu.VMEM((B, SWP), jnp.int32),
          pltpu.VMEM((B, SWP), jnp.int32),
          pltpu.VMEM((B, SWP), jnp.float32),
          pltpu.VMEM((B, SWP), jnp.float32),
          pltpu.VMEM((B, SWP), jnp.int32),
          pltpu.VMEM((B, SWP), jnp.int32),
          pltpu.VMEM((B, SWP), jnp.float32),
          pltpu.VMEM((B, SWP), jnp.float32),
          pltpu.VMEM((2 * NP,), jnp.float32),
          pltpu.VMEM((2 * NP,), jnp.float32),
          pltpu.VMEM((2 * NP,), jnp.float32),
          pltpu.VMEM((2 * NP,), jnp.float32),
          pltpu.VMEM((8 * SLICE,), jnp.float32),
          pltpu.VMEM((8 * SLICE,), jnp.float32),
          pltpu.VMEM((SLICE,), jnp.float32),
          pltpu.VMEM((4 * SLICE,), jnp.float32),
          pltpu.VMEM((MSE_CHUNK,), jnp.float32),
          pltpu.VMEM((MSE_CHUNK,), jnp.float32),
          pltpu.VMEM((MSE_CHUNK,), jnp.float32),
          pltpu.VMEM((MSE_CHUNK,), jnp.float32),
          pltpu.VMEM((L,), jnp.float32),
          pltpu.VMEM_SHARED((4 * NP,), jnp.float32),
          pltpu.SemaphoreType.DMA,
          pltpu.SemaphoreType.DMA,
          pltpu.SemaphoreType.DMA,
          pltpu.SemaphoreType.DMA,
          pltpu.SemaphoreType.DMA,
          pltpu.SemaphoreType.DMA,
          pltpu.SemaphoreType.DMA,
          pltpu.SemaphoreType.DMA,
      ],
  )
  return f(er, ei, ymr, ymi, rows, cols, vr, vi, br, bi)


def kernel(E_real, E_imag, batch_y, k_all, node_batch, A_rows, A_cols,
           A_vals_real, A_vals_imag, b_real, b_imag):
  del k_all, node_batch
  partials = _run(E_real, E_imag, batch_y[:, 0], batch_y[:, 1],
                  A_rows, A_cols, A_vals_real, A_vals_imag,
                  b_real.reshape(-1), b_imag.reshape(-1))
  return jnp.sum(partials)


# speedup vs baseline: 1.0622x; 1.0100x over previous
import functools

import jax
import jax.numpy as jnp
from jax import lax
from jax.experimental import pallas as pl
from jax.experimental.pallas import tpu as pltpu
from jax.experimental.pallas import tpu_sc as plsc

B = 4
NP = 10000
NNZ = 160000
N = B * NP

NC = 2
NS = 16
L = 16

BLK = 128
NB = NNZ // BLK
NB_TILE = 78
KP = 2
SWP = KP * BLK
NPASS = 39
NRING = 4
TAIL_OFF = NS * NB_TILE * BLK
TAIL_W = NNZ - TAIL_OFF
NP_PAD = 10240
SLICE = NP_PAD // NS
MSE_CHUNK = 1280
NV = SWP // L


def _sc_body(er_hbm, ei_hbm, ymr_hbm, ymi_hbm, rows_hbm, cols_hbm,
             vr_hbm, vi_hbm, br_hbm, bi_hbm,
             out_hbm, exch_hbm,
             str0, stc0, stvr0, stvi0, str1, stc1, stvr1, stvi1,
             str2, stc2, stvr2, stvi2, str3, stc3, stvr3, stvi3,
             accr, acci, xr2, xi2,
             tmpa, tmpb, axbuf, b4, m2, m3, outv,
             sem_st0, sem_st1, sem_st2, sem_st3, sem_x, sem_b, sem_mse,
             sem_t, sem_t2):
  c = lax.axis_index("c")
  s = lax.axis_index("s")

  zeros = jnp.zeros((L,), jnp.float32)
  iota = lax.broadcasted_iota(jnp.int32, (L,), 0)
  wid = c * NS + s

  st = [(str0, stc0, stvr0, stvi0, sem_st0),
        (str1, stc1, stvr1, stvi1, sem_st1),
        (str2, stc2, stvr2, stvi2, sem_st2),
        (str3, stc3, stvr3, stvi3, sem_st3)]

  x_copies = [
      pltpu.async_copy(er_hbm.at[pl.ds(c * 2 * NP, 2 * NP)], xr2, sem_x),
      pltpu.async_copy(ei_hbm.at[pl.ds(c * 2 * NP, 2 * NP)], xi2, sem_x),
  ]
  mse_start = 2 * c * NP + 1250 * s
  y_off = pl.multiple_of(
      jnp.minimum((mse_start // 8) * 8, N - MSE_CHUNK), 8)
  y_shift = mse_start - y_off
  mse_copies = [
      pltpu.async_copy(ymr_hbm.at[pl.ds(y_off, MSE_CHUNK)], m2, sem_mse),
      pltpu.async_copy(ymi_hbm.at[pl.ds(y_off, MSE_CHUNK)], m3, sem_mse),
  ]

  col0 = s * (NB_TILE * BLK)

  def pass_copies(p, parity):
    r, co, vr_, vi_, sem = st[parity]
    off = pl.multiple_of(col0 + p * SWP, BLK)
    return [
        pltpu.make_async_copy(rows_hbm.at[:, pl.ds(off, SWP)], r, sem),
        pltpu.make_async_copy(cols_hbm.at[:, pl.ds(off, SWP)], co, sem),
        pltpu.make_async_copy(vr_hbm.at[:, pl.ds(off, SWP)], vr_, sem),
        pltpu.make_async_copy(vi_hbm.at[:, pl.ds(off, SWP)], vi_, sem),
    ]

  def fire_pass(p, parity):
    for cp in pass_copies(p, parity):
      cp.start()

  def wait_pass(p, parity):
    for cp in pass_copies(p, parity):
      cp.wait()

  fire_pass(0, 0)
  fire_pass(1, 1)
  fire_pass(2, 2)

  @plsc.parallel_loop(0, 2 * NP // L, unroll=5)
  def _(k):
    off = pl.ds(k * L, L)
    accr[off] = zeros
    acci[off] = zeros

  with jax.named_scope("x_wait"):
    for cp in x_copies:
      cp.wait()

  brow16_0 = jnp.full((L,), 2 * c, jnp.int32)

  def phase1_block(r_ref, c_ref, vr_ref, vi_ref, nvregs):
    @plsc.parallel_loop(0, 2 * nvregs, unroll=4)
    def _(t):
      second = t >= nvregs
      idx16 = jnp.where(second, t - nvregs, t) * L + iota
      xoff = jnp.where(second, NP, 0)
      brow16 = brow16_0 + second.astype(jnp.int32)
      rowg = plsc.load_gather(r_ref, [brow16, idx16])
      colg = plsc.load_gather(c_ref, [brow16, idx16])
      wr = plsc.load_gather(vr_ref, [brow16, idx16])
      wi = plsc.load_gather(vi_ref, [brow16, idx16])
      xcr = plsc.load_gather(xr2, [colg + xoff])
      xci = plsc.load_gather(xi2, [colg + xoff])
      ar = wr * xcr - wi * xci
      ai = wr * xci + wi * xcr
      plsc.addupdate_scatter(accr, [rowg + xoff], ar)
      plsc.addupdate_scatter(acci, [rowg + xoff], ai)

  def pass_quad(k, _):
    p0 = k * 4
    for j in range(NRING):
      p = p0 + j

      @pl.when(p + 3 < NPASS)
      def _():
        fire_pass(p + 3, (j + 3) % NRING)

      with jax.named_scope("st_wait"):
        wait_pass(p, j)
      r_ref, c_ref, vr_ref, vi_ref, _ = st[j]
      with jax.named_scope("phase1"):
        phase1_block(r_ref, c_ref, vr_ref, vi_ref, NV)
    return 0

  lax.fori_loop(0, 36 // NRING, pass_quad, 0)
  for p in (36, 37, 38):
    with jax.named_scope("st_wait"):
      wait_pass(p, p % NRING)
    r_ref, c_ref, vr_ref, vi_ref, _ = st[p % NRING]
    with jax.named_scope("phase1"):
      phase1_block(r_ref, c_ref, vr_ref, vi_ref, NV)

  @pl.when(s == 0)
  def _():
    pltpu.sync_copy(rows_hbm.at[:, pl.ds(TAIL_OFF, TAIL_W)], str0)
    pltpu.sync_copy(cols_hbm.at[:, pl.ds(TAIL_OFF, TAIL_W)], stc0)
    pltpu.sync_copy(vr_hbm.at[:, pl.ds(TAIL_OFF, TAIL_W)], stvr0)
    pltpu.sync_copy(vi_hbm.at[:, pl.ds(TAIL_OFF, TAIL_W)], stvi0)
    phase1_block(str0, stc0, stvr0, stvi0, TAIL_W // L)

  row_base = s * SLICE
  b_off = pl.multiple_of(jnp.minimum(row_base, NP - SLICE), 8)
  b_shift = row_base - b_off
  b_srcs = [br_hbm.at[pl.ds(pl.multiple_of(2 * c * NP, 8) + b_off, SLICE)],
            bi_hbm.at[pl.ds(pl.multiple_of(2 * c * NP, 8) + b_off, SLICE)],
            br_hbm.at[pl.ds(pl.multiple_of((2 * c + 1) * NP, 8) + b_off,
                            SLICE)],
            bi_hbm.at[pl.ds(pl.multiple_of((2 * c + 1) * NP, 8) + b_off,
                            SLICE)]]
  bcp = [pltpu.async_copy(src, b4.at[pl.ds(q * SLICE, SLICE)], sem_b)
         for q, src in enumerate(b_srcs)]

  def pub_off(tile, slot):
    return pl.multiple_of(((c * NS + tile) * 4 + slot) * NP, 8)

  with jax.named_scope("publish"):
    pubs0 = [
        pltpu.async_copy(accr.at[pl.ds(0, NP)],
                         exch_hbm.at[pl.ds(pub_off(s, 0), NP)], sem_x),
        pltpu.async_copy(acci.at[pl.ds(0, NP)],
                         exch_hbm.at[pl.ds(pub_off(s, 1), NP)], sem_x),
    ]
    pubs1 = [
        pltpu.async_copy(accr.at[pl.ds(NP, NP)],
                         exch_hbm.at[pl.ds(pub_off(s, 2), NP)], sem_st0),
        pltpu.async_copy(acci.at[pl.ds(NP, NP)],
                         exch_hbm.at[pl.ds(pub_off(s, 3), NP)], sem_st0),
    ]
    for cp in pubs0:
      cp.wait()
    plsc.subcore_barrier()

  def half_copies(q, h, buf, sem):
    return [
        pltpu.make_async_copy(
            exch_hbm.at[pl.ds(
                pl.multiple_of(pub_off(h * 8 + j, q) + b_off, 8), SLICE)],
            buf.at[pl.ds(j * SLICE, SLICE)], sem)
        for j in range(8)
    ]

  def fire_half(q, h, buf, sem):
    for cp in half_copies(q, h, buf, sem):
      cp.start()

  def wait_half(q, h, buf, sem):
    for cp in half_copies(q, h, buf, sem):
      cp.wait()

  fire_half(0, 0, tmpa, sem_t)
  fire_half(0, 1, tmpb, sem_t2)

  loss_vec = zeros

  def slot_round(q, acc):
    with jax.named_scope("tmp_wait"):
      wait_half(q, 0, tmpa, sem_t)

    @plsc.parallel_loop(0, SLICE // L)
    def half0(k):
      koff = jnp.minimum(b_shift + k * L, SLICE - L)
      part = tmpa[pl.ds(koff, L)]
      for j in range(1, 8):
        part = part + tmpa[pl.ds(j * SLICE + koff, L)]
      axbuf[pl.ds(k * L, L)] = part

    @pl.when(q < 3)
    def _():
      fire_half(q + 1, 0, tmpa, sem_t)

    with jax.named_scope("tmp_wait"):
      wait_half(q, 1, tmpb, sem_t2)

    @plsc.parallel_loop(0, SLICE // L, carry=acc)
    def half1(k, a):
      koff = jnp.minimum(b_shift + k * L, SLICE - L)
      part = tmpb[pl.ds(koff, L)]
      for j in range(1, 8):
        part = part + tmpb[pl.ds(j * SLICE + koff, L)]
      ax = axbuf[pl.ds(k * L, L)] + part
      r = b4[pl.ds(q * SLICE + koff, L)] - ax
      row_vec = row_base + k * L + iota
      return a + jnp.where(row_vec < NP, r * r, jnp.float32(0.0))

    @pl.when(q < 3)
    def _():
      fire_half(q + 1, 1, tmpb, sem_t2)

    return half1

  with jax.named_scope("b_wait"):
    for cp in bcp:
      cp.wait()
  loss_vec = slot_round(0, loss_vec)
  with jax.named_scope("publish"):
    for cp in pubs1:
      cp.wait()
    plsc.subcore_barrier()
  loss_vec = slot_round(1, loss_vec)
  loss_vec = slot_round(2, loss_vec)
  loss_vec = slot_round(3, loss_vec)

  with jax.named_scope("mse_wait"):
    for cp in mse_copies:
      cp.wait()

  xbase = 1250 * s

  @plsc.parallel_loop(0, 79, carry=loss_vec)
  def mse_body(k, acc):
    eoff = jnp.minimum(k * L, 1250 - L)
    lane = eoff + iota
    mask = (lane >= k * L) & (lane < 1250)
    dr = xr2[pl.ds(xbase + eoff, L)] - m2[pl.ds(y_shift + eoff, L)]
    di = xi2[pl.ds(xbase + eoff, L)] - m3[pl.ds(y_shift + eoff, L)]
    contrib = dr * dr + di * di
    return acc + jnp.where(mask, contrib, jnp.float32(0.0))
  loss_vec = mse_body

  outv[...] = loss_vec * jnp.float32(0.5 / N)
  pltpu.sync_copy(outv, out_hbm.at[wid])


@jax.jit
def _run(er, ei, ymr, ymi, rows, cols, vr, vi, br, bi):
  mesh = plsc.VectorSubcoreMesh(
      core_axis_name="c", subcore_axis_name="s",
      num_cores=NC, num_subcores=NS)
  f = pl.kernel(
      _sc_body,
      out_type=jax.ShapeDtypeStruct((NC * NS, L), jnp.float32),
      mesh=mesh,
      compiler_params=pltpu.CompilerParams(
          needs_layout_passes=False, skip_device_barrier=True,
          disable_bounds_checks=True),
      scratch_types=[
          pltpu.HBM((NC * NS * 4 * NP,), jnp.float32),
          pltpu.VMEM((B, SWP), jnp.int32),
          pltpu.VMEM((B, SWP), jnp.int32),
          pltpu.VMEM((B, SWP), jnp.float32),
          pltpu.VMEM((B, SWP), jnp.float32),
          pltpu.VMEM((B, SWP), jnp.int32),
          pltpu.VMEM((B, SWP), jnp.int32),
          pltpu.VMEM((B, SWP), jnp.float32),
          pltpu.VMEM((B, SWP), jnp.float32),
          pltpu.VMEM((B, SWP), jnp.int32),
          pltpu.VMEM((B, SWP), jnp.int32),
          pltpu.VMEM((B, SWP), jnp.float32),
          pltpu.VMEM((B, SWP), jnp.float32),
          pltpu.VMEM((B, SWP), jnp.int32),
          pltpu.VMEM((B, SWP), jnp.int32),
          pltpu.VMEM((B, SWP), jnp.float32),
          pltpu.VMEM((B, SWP), jnp.float32),
          pltpu.VMEM((2 * NP,), jnp.float32),
          pltpu.VMEM((2 * NP,), jnp.float32),
          pltpu.VMEM((2 * NP,), jnp.float32),
          pltpu.VMEM((2 * NP,), jnp.float32),
          pltpu.VMEM((8 * SLICE,), jnp.float32),
          pltpu.VMEM((8 * SLICE,), jnp.float32),
          pltpu.VMEM((SLICE,), jnp.float32),
          pltpu.VMEM((4 * SLICE,), jnp.float32),
          pltpu.VMEM((MSE_CHUNK,), jnp.float32),
          pltpu.VMEM((MSE_CHUNK,), jnp.float32),
          pltpu.VMEM((L,), jnp.float32),
          pltpu.SemaphoreType.DMA,
          pltpu.SemaphoreType.DMA,
          pltpu.SemaphoreType.DMA,
          pltpu.SemaphoreType.DMA,
          pltpu.SemaphoreType.DMA,
          pltpu.SemaphoreType.DMA,
          pltpu.SemaphoreType.DMA,
          pltpu.SemaphoreType.DMA,
          pltpu.SemaphoreType.DMA,
      ],
  )
  return f(er, ei, ymr, ymi, rows, cols, vr, vi, br, bi)


def kernel(E_real, E_imag, batch_y, k_all, node_batch, A_rows, A_cols,
           A_vals_real, A_vals_imag, b_real, b_imag):
  del k_all, node_batch
  partials = _run(E_real, E_imag, batch_y[:, 0], batch_y[:, 1],
                  A_rows, A_cols, A_vals_real, A_vals_imag,
                  b_real.reshape(-1), b_imag.reshape(-1))
  return jnp.sum(partials)
